# baseline (speedup 1.0000x reference)
import jax
import jax.numpy as jnp
from jax.experimental import pallas as pl
from jax.experimental.pallas import tpu as pltpu

B, H, S, D, Q = 16, 8, 2048, 128, 1
G = 4


def _body(pos_ref, k_ref, v_ref, lk_ref, lv_ref, ok_ref, ov_ref):
    g = pl.program_id(0)
    ok_ref[...] = k_ref[...]
    ov_ref[...] = v_ref[...]
    for j in range(G):
        local = pos_ref[g * G + j]
        ok_ref[j, pl.ds(local, 1), :] = lk_ref[j]
        ov_ref[j, pl.ds(local, 1), :] = lv_ref[j]


def kernel(k_cache, v_cache, latest_k, latest_v, position_ids):
    BH = B * H
    pos_pair = jnp.repeat(position_ids.reshape(B).astype(jnp.int32), H)
    k2 = k_cache.reshape(BH, S, D)
    v2 = v_cache.reshape(BH, S, D)
    lk2 = latest_k.reshape(BH, Q, D)
    lv2 = latest_v.reshape(BH, Q, D)
    grid_spec = pltpu.PrefetchScalarGridSpec(
        num_scalar_prefetch=1,
        grid=(BH // G,),
        in_specs=[
            pl.BlockSpec((G, S, D), lambda g, p: (g, 0, 0)),
            pl.BlockSpec((G, S, D), lambda g, p: (g, 0, 0)),
            pl.BlockSpec((G, Q, D), lambda g, p: (g, 0, 0)),
            pl.BlockSpec((G, Q, D), lambda g, p: (g, 0, 0)),
        ],
        out_specs=[
            pl.BlockSpec((G, S, D), lambda g, p: (g, 0, 0)),
            pl.BlockSpec((G, S, D), lambda g, p: (g, 0, 0)),
        ],
    )
    out_shape = [
        jax.ShapeDtypeStruct((BH, S, D), k_cache.dtype),
        jax.ShapeDtypeStruct((BH, S, D), v_cache.dtype),
    ]
    k_new, v_new = pl.pallas_call(
        _body,
        grid_spec=grid_spec,
        out_shape=out_shape,
    )(pos_pair, k2, v2, lk2, lv2)
    return (k_new.reshape(B, H, S, D), v_new.reshape(B, H, S, D))
